# baseline (speedup 1.0000x reference)
import functools

import jax
import jax.numpy as jnp
from jax import lax
from jax.experimental import pallas as pl
from jax.experimental.pallas import tpu as pltpu
from jax.experimental.pallas import tpu_sc as plsc

B = 16384
D = 256
NC = 2
NS = 16
NW = NC * NS
BPW = B // NW
CHUNK = 128
NCHUNK = BPW // CHUNK

_mesh = plsc.VectorSubcoreMesh(core_axis_name="c", subcore_axis_name="s")


@functools.partial(
    pl.kernel,
    mesh=_mesh,
    out_type=jax.ShapeDtypeStruct((B, D), jnp.float32),
    scratch_types=[
        pltpu.VMEM((NW, BPW), jnp.int32),
        pltpu.VMEM((CHUNK, D), jnp.float32),
        pltpu.VMEM((CHUNK, D), jnp.float32),
        pltpu.SemaphoreType.DMA,
        pltpu.SemaphoreType.DMA,
    ],
)
def _lookup(idx_hbm, table_hbm, out_hbm, idx_v, buf0, buf1, sem0, sem1):
    wid = lax.axis_index("s") * NC + lax.axis_index("c")
    base = wid * BPW
    my_idx = idx_v.at[wid]
    pltpu.sync_copy(idx_hbm.at[wid], my_idx)
    bufs = (buf0, buf1)
    sems = (sem0, sem1)
    for ch in range(NCHUNK):
        b = bufs[ch % 2]
        s = sems[ch % 2]
        pltpu.async_copy(
            table_hbm.at[my_idx.at[pl.ds(ch * CHUNK, CHUNK)]], b, s
        ).wait()
        pltpu.sync_copy(b, out_hbm.at[pl.ds(base + ch * CHUNK, CHUNK)])


def kernel(harmful, emb_weight):
    idx = jnp.reshape(harmful.astype(jnp.int32), (NW, BPW))
    return _lookup(idx, emb_weight)


# speedup vs baseline: 9.2217x; 9.2217x over previous
import functools

import jax
import jax.numpy as jnp
from jax import lax
from jax.experimental import pallas as pl
from jax.experimental.pallas import tpu as pltpu
from jax.experimental.pallas import tpu_sc as plsc

B = 16384
D = 256
L = 16
NLC = D // L
NC = 2
NS = 16
NW = NC * NS
BPW = B // NW
CHUNK = 128
NCHUNK = BPW // CHUNK

_mesh = plsc.VectorSubcoreMesh(core_axis_name="c", subcore_axis_name="s")


@functools.partial(
    pl.kernel,
    mesh=_mesh,
    out_type=jax.ShapeDtypeStruct((B, D), jnp.float32),
    scratch_types=[
        pltpu.VMEM((BPW,), jnp.int32),
        pltpu.VMEM((2, D), jnp.float32),
        pltpu.VMEM((CHUNK, D), jnp.float32),
        pltpu.VMEM((CHUNK, D), jnp.float32),
        pltpu.SemaphoreType.DMA,
        pltpu.SemaphoreType.DMA,
    ],
)
def _lookup(idx_hbm, table_hbm, out_hbm, idx_v, table_v, buf0, buf1,
            sem0, sem1):
    wid = lax.axis_index("s") * NC + lax.axis_index("c")
    base = wid * BPW
    pltpu.sync_copy(table_hbm, table_v)
    pltpu.sync_copy(idx_hbm.at[wid], idx_v)
    w0 = [table_v[0, pl.ds(c * L, L)] for c in range(NLC)]
    w1 = [table_v[1, pl.ds(c * L, L)] for c in range(NLC)]

    bufs = (buf0, buf1)
    sems = (sem0, sem1)

    def fill_chunk(ch, buf):
        def grp(g, carry):
            iv = idx_v[pl.ds(ch * CHUNK + g * L, L)]
            for l in range(L):
                take1 = iv[l] != 0
                for c in range(NLC):
                    buf[g * L + l, pl.ds(c * L, L)] = jnp.where(
                        take1, w1[c], w0[c])
            return carry
        lax.fori_loop(0, CHUNK // L, grp, 0)

    copies = [None] * NCHUNK
    for ch in range(NCHUNK):
        if ch >= 2:
            copies[ch - 2].wait()
        fill_chunk(ch, bufs[ch % 2])
        copies[ch] = pltpu.async_copy(
            bufs[ch % 2], out_hbm.at[pl.ds(base + ch * CHUNK, CHUNK)],
            sems[ch % 2])
    for ch in range(max(0, NCHUNK - 2), NCHUNK):
        copies[ch].wait()


def kernel(harmful, emb_weight):
    idx = jnp.reshape(harmful.astype(jnp.int32), (NW, BPW))
    return _lookup(idx, emb_weight)


# speedup vs baseline: 13.4134x; 1.4545x over previous
import functools

import jax
import jax.numpy as jnp
from jax import lax
from jax.experimental import pallas as pl
from jax.experimental.pallas import tpu as pltpu
from jax.experimental.pallas import tpu_sc as plsc

B = 16384
D = 256
L = 16
NC = 2
NS = 16
NW = NC * NS
BPW = B // NW

_mesh = plsc.VectorSubcoreMesh(core_axis_name="c", subcore_axis_name="s")


@functools.partial(
    pl.kernel,
    mesh=_mesh,
    out_type=jax.ShapeDtypeStruct((B, D), jnp.float32),
    scratch_types=[
        pltpu.VMEM((BPW,), jnp.int32),
        pltpu.VMEM((2, D), jnp.float32),
        pltpu.SemaphoreType.DMA,
    ],
)
def _lookup(idx_hbm, table_hbm, out_hbm, idx_v, table_v, sem):
    wid = lax.axis_index("s") * NC + lax.axis_index("c")
    base = wid * BPW
    pltpu.sync_copy(table_hbm, table_v)
    pltpu.sync_copy(idx_hbm.at[wid], idx_v)

    def grp(g, carry):
        iv = idx_v[pl.ds(g * L, L)]
        for l in range(L):
            r = iv[l]
            pltpu.async_copy(
                table_v.at[pl.ds(r, 1)],
                out_hbm.at[pl.ds(base + g * L + l, 1)],
                sem)
        return carry

    lax.fori_loop(0, BPW // L, grp, 0)
    my_out = out_hbm.at[pl.ds(base, BPW)]
    pltpu.make_async_copy(my_out, my_out, sem).wait()


def kernel(harmful, emb_weight):
    idx = jnp.reshape(harmful.astype(jnp.int32), (NW, BPW))
    return _lookup(idx, emb_weight)
